# initial kernel scaffold (unmeasured)
import jax
import jax.numpy as jnp
from jax import lax
from jax.experimental import pallas as pl
from jax.experimental.pallas import tpu as pltpu

N_DEV = 4


def kernel(x, W1, W2):
    x = x.astype(jnp.bfloat16)
    W1 = W1.astype(jnp.bfloat16)
    W2 = W2.astype(jnp.bfloat16)
    m, _ = x.shape
    out_n = W2.shape[1]

    def body(x_ref, w1_ref, w2_ref, out_ref, comm_ref, send_sems, recv_sems):
        my_pos = lax.axis_index("i")
        left = (my_pos + N_DEV - 1) % N_DEV
        right = (my_pos + 1) % N_DEV

        barrier_sem = pltpu.get_barrier_semaphore()
        for nbr in [left, right]:
            pl.semaphore_signal(
                barrier_sem, inc=1,
                device_id=(nbr,), device_id_type=pl.DeviceIdType.MESH,
            )
        pl.semaphore_wait(barrier_sem, 2)

        h = jnp.dot(x_ref[...], w1_ref[...], preferred_element_type=jnp.float32)
        h = jnp.maximum(h, 0.0).astype(jnp.bfloat16)
        partial = jnp.dot(h, w2_ref[...], preferred_element_type=jnp.float32)

        acc = partial
        comm_ref[0, :, :] = partial.astype(jnp.bfloat16)

        for hop in range(N_DEV - 1):
            send_slot = hop % 2
            recv_slot = (hop + 1) % 2
            rdma = pltpu.make_async_remote_copy(
                src_ref=comm_ref.at[send_slot],
                dst_ref=comm_ref.at[recv_slot],
                send_sem=send_sems.at[send_slot],
                recv_sem=recv_sems.at[recv_slot],
                device_id=(right,),
                device_id_type=pl.DeviceIdType.MESH,
            )
            rdma.start()
            rdma.wait()
            acc = acc + comm_ref[recv_slot, :, :].astype(jnp.float32)

        out_ref[...] = acc

    return pl.pallas_call(
        body,
        out_shape=jax.ShapeDtypeStruct((m, out_n), jnp.float32),
        in_specs=[
            pl.BlockSpec(memory_space=pltpu.VMEM),
            pl.BlockSpec(memory_space=pltpu.VMEM),
            pl.BlockSpec(memory_space=pltpu.VMEM),
        ],
        out_specs=pl.BlockSpec(memory_space=pltpu.VMEM),
        scratch_shapes=[
            pltpu.VMEM((2, m, out_n), jnp.bfloat16),
            pltpu.SemaphoreType.DMA((2,)),
            pltpu.SemaphoreType.DMA((2,)),
        ],
        compiler_params=pltpu.CompilerParams(collective_id=0),
    )(x, W1, W2)


# baseline (device time: 236259 ns/iter reference)
import jax
import jax.numpy as jnp
from jax import lax
from jax.experimental import pallas as pl
from jax.experimental.pallas import tpu as pltpu

N_DEV = 4


def kernel(x, W1, W2):
    x = x.astype(jnp.bfloat16)
    W1 = W1.astype(jnp.bfloat16)
    W2 = W2.astype(jnp.bfloat16)
    m, _ = x.shape
    out_n = W2.shape[1]

    def body(x_ref, w1_ref, w2_ref, out_ref, comm_ref, send_sems, recv_sems):
        my_pos = lax.axis_index("i")
        left = (my_pos + N_DEV - 1) % N_DEV
        right = (my_pos + 1) % N_DEV

        barrier_sem = pltpu.get_barrier_semaphore()
        for nbr in [left, right]:
            pl.semaphore_signal(
                barrier_sem, inc=1,
                device_id=(nbr,), device_id_type=pl.DeviceIdType.MESH,
            )
        pl.semaphore_wait(barrier_sem, 2)

        h = jnp.dot(x_ref[...], w1_ref[...], preferred_element_type=jnp.float32)
        h = jnp.maximum(h, 0.0).astype(jnp.bfloat16)
        partial = jnp.dot(h, w2_ref[...], preferred_element_type=jnp.float32)

        acc = partial
        comm_ref[0, :, :] = partial.astype(jnp.bfloat16)

        for hop in range(N_DEV - 1):
            send_slot = hop % 2
            recv_slot = (hop + 1) % 2
            rdma = pltpu.make_async_remote_copy(
                src_ref=comm_ref.at[send_slot],
                dst_ref=comm_ref.at[recv_slot],
                send_sem=send_sems.at[send_slot],
                recv_sem=recv_sems.at[recv_slot],
                device_id=(right,),
                device_id_type=pl.DeviceIdType.MESH,
            )
            rdma.start()
            rdma.wait()
            acc = acc + comm_ref[recv_slot, :, :].astype(jnp.float32)

        out_ref[...] = acc

    return pl.pallas_call(
        body,
        out_shape=jax.ShapeDtypeStruct((m, out_n), jnp.float32),
        in_specs=[
            pl.BlockSpec(memory_space=pltpu.VMEM),
            pl.BlockSpec(memory_space=pltpu.VMEM),
            pl.BlockSpec(memory_space=pltpu.VMEM),
        ],
        out_specs=pl.BlockSpec(memory_space=pltpu.VMEM),
        scratch_shapes=[
            pltpu.VMEM((2, m, out_n), jnp.bfloat16),
            pltpu.SemaphoreType.DMA((2,)),
            pltpu.SemaphoreType.DMA((2,)),
        ],
        compiler_params=pltpu.CompilerParams(
            collective_id=0,
            vmem_limit_bytes=128 * 1024 * 1024,
        ),
    )(x, W1, W2)


# device time: 140612 ns/iter; 1.6802x vs baseline; 1.6802x over previous
import jax
import jax.numpy as jnp
from jax import lax
from jax.experimental import pallas as pl
from jax.experimental.pallas import tpu as pltpu

N_DEV = 4


def kernel(x, W1, W2):
    x = x.astype(jnp.bfloat16)
    W1 = W1.astype(jnp.bfloat16)
    W2 = W2.astype(jnp.bfloat16)
    m, _ = x.shape
    out_n = W2.shape[1]
    ch = m // N_DEV

    def body(x_ref, w1_ref, w2_ref, out_ref, comm_ref, send_sems, recv_sems):
        my_pos = lax.axis_index("i")
        left = (my_pos + N_DEV - 1) % N_DEV
        right = (my_pos + 1) % N_DEV

        barrier_sem = pltpu.get_barrier_semaphore()
        for nbr in [left, right]:
            pl.semaphore_signal(
                barrier_sem, inc=1,
                device_id=(nbr,), device_id_type=pl.DeviceIdType.MESH,
            )
        pl.semaphore_wait(barrier_sem, 2)

        def compute_chunk(c):
            xs = x_ref[pl.ds(c * ch, ch), :]
            h = jnp.dot(xs, w1_ref[...], preferred_element_type=jnp.float32)
            h = jnp.maximum(h, 0.0).astype(jnp.bfloat16)
            return jnp.dot(h, w2_ref[...], preferred_element_type=jnp.float32)

        def hop(k):
            send_slot = k % 2
            recv_slot = (k + 1) % 2
            return pltpu.make_async_remote_copy(
                src_ref=comm_ref.at[send_slot],
                dst_ref=comm_ref.at[recv_slot],
                send_sem=send_sems.at[send_slot],
                recv_sem=recv_sems.at[recv_slot],
                device_id=(right,),
                device_id_type=pl.DeviceIdType.MESH,
            )

        comm_ref[0, :, :] = compute_chunk(my_pos).astype(jnp.bfloat16)
        for k in range(N_DEV - 1):
            rdma = hop(k)
            rdma.start()
            c_next = (my_pos + (N_DEV - 1 - k)) % N_DEV
            p_next = compute_chunk(c_next)
            rdma.wait()
            recv_slot = (k + 1) % 2
            acc = comm_ref[recv_slot, :, :].astype(jnp.float32) + p_next
            comm_ref[recv_slot, :, :] = acc.astype(jnp.bfloat16)
            if k == N_DEV - 2:
                out_ref[pl.ds(c_next * ch, ch), :] = acc

        for k in range(N_DEV - 1, 2 * (N_DEV - 1)):
            rdma = hop(k)
            rdma.start()
            rdma.wait()
            recv_slot = (k + 1) % 2
            t = k - (N_DEV - 1)
            c_recv = (my_pos + (N_DEV - t)) % N_DEV
            out_ref[pl.ds(c_recv * ch, ch), :] = (
                comm_ref[recv_slot, :, :].astype(jnp.float32)
            )

    return pl.pallas_call(
        body,
        out_shape=jax.ShapeDtypeStruct((m, out_n), jnp.float32),
        in_specs=[
            pl.BlockSpec(memory_space=pltpu.VMEM),
            pl.BlockSpec(memory_space=pltpu.VMEM),
            pl.BlockSpec(memory_space=pltpu.VMEM),
        ],
        out_specs=pl.BlockSpec(memory_space=pltpu.VMEM),
        scratch_shapes=[
            pltpu.VMEM((2, ch, out_n), jnp.bfloat16),
            pltpu.SemaphoreType.DMA((2,)),
            pltpu.SemaphoreType.DMA((2,)),
        ],
        compiler_params=pltpu.CompilerParams(
            collective_id=0,
            vmem_limit_bytes=128 * 1024 * 1024,
        ),
    )(x, W1, W2)


# device time: 100609 ns/iter; 2.3483x vs baseline; 1.3976x over previous
import jax
import jax.numpy as jnp
from jax import lax
from jax.experimental import pallas as pl
from jax.experimental.pallas import tpu as pltpu

N_DEV = 4


def kernel(x, W1, W2):
    W1 = W1.astype(jnp.bfloat16)
    W2 = W2.astype(jnp.bfloat16)
    m, _ = x.shape
    out_n = W2.shape[1]
    ch = m // N_DEV
    half = ch // 2

    def body(x_ref, w1_ref, w2_ref, out_ref,
             comm_r, comm_l,
             send_r, recv_r, send_l, recv_l):
        my_pos = lax.axis_index("i")
        left = (my_pos + N_DEV - 1) % N_DEV
        right = (my_pos + 1) % N_DEV

        def compute_half(c, off):
            xs = x_ref[pl.ds(c * ch + off, half), :].astype(jnp.bfloat16)
            h = jnp.dot(xs, w1_ref[...], preferred_element_type=jnp.float32)
            h = jnp.maximum(h, 0.0).astype(jnp.bfloat16)
            return jnp.dot(h, w2_ref[...], preferred_element_type=jnp.float32)

        comm_r[0, :, :] = compute_half(my_pos, 0).astype(jnp.bfloat16)
        comm_l[0, :, :] = compute_half(my_pos, half).astype(jnp.bfloat16)

        barrier_sem = pltpu.get_barrier_semaphore()
        for nbr in [left, right]:
            pl.semaphore_signal(
                barrier_sem, inc=1,
                device_id=(nbr,), device_id_type=pl.DeviceIdType.MESH,
            )
        pl.semaphore_wait(barrier_sem, 2)

        def hops(k):
            ss, rs = k % 2, (k + 1) % 2
            rdma_r = pltpu.make_async_remote_copy(
                src_ref=comm_r.at[ss], dst_ref=comm_r.at[rs],
                send_sem=send_r.at[ss], recv_sem=recv_r.at[rs],
                device_id=(right,), device_id_type=pl.DeviceIdType.MESH,
            )
            rdma_l = pltpu.make_async_remote_copy(
                src_ref=comm_l.at[ss], dst_ref=comm_l.at[rs],
                send_sem=send_l.at[ss], recv_sem=recv_l.at[rs],
                device_id=(left,), device_id_type=pl.DeviceIdType.MESH,
            )
            return rdma_r, rdma_l

        for k in range(N_DEV - 1):
            rdma_r, rdma_l = hops(k)
            rdma_r.start()
            rdma_l.start()
            c_r = (my_pos + (N_DEV - 1 - k)) % N_DEV
            c_l = (my_pos + k + 1) % N_DEV
            p_r = compute_half(c_r, 0)
            p_l = compute_half(c_l, half)
            rdma_r.wait()
            rdma_l.wait()
            rs = (k + 1) % 2
            acc_r = comm_r[rs, :, :].astype(jnp.float32) + p_r
            acc_l = comm_l[rs, :, :].astype(jnp.float32) + p_l
            comm_r[rs, :, :] = acc_r.astype(jnp.bfloat16)
            comm_l[rs, :, :] = acc_l.astype(jnp.bfloat16)
            if k == N_DEV - 2:
                out_ref[pl.ds(c_r * ch, half), :] = acc_r
                out_ref[pl.ds(c_l * ch + half, half), :] = acc_l

        for k in range(N_DEV - 1, 2 * (N_DEV - 1)):
            t = k - (N_DEV - 1)
            rdma_r, rdma_l = hops(k)
            rdma_r.start()
            rdma_l.start()
            rdma_r.wait()
            rdma_l.wait()
            rs = (k + 1) % 2
            c_r = (my_pos + (N_DEV - t)) % N_DEV
            c_l = (my_pos + t) % N_DEV
            out_ref[pl.ds(c_r * ch, half), :] = (
                comm_r[rs, :, :].astype(jnp.float32))
            out_ref[pl.ds(c_l * ch + half, half), :] = (
                comm_l[rs, :, :].astype(jnp.float32))

    return pl.pallas_call(
        body,
        out_shape=jax.ShapeDtypeStruct((m, out_n), jnp.float32),
        in_specs=[
            pl.BlockSpec(memory_space=pltpu.VMEM),
            pl.BlockSpec(memory_space=pltpu.VMEM),
            pl.BlockSpec(memory_space=pltpu.VMEM),
        ],
        out_specs=pl.BlockSpec(memory_space=pltpu.VMEM),
        scratch_shapes=[
            pltpu.VMEM((2, half, out_n), jnp.bfloat16),
            pltpu.VMEM((2, half, out_n), jnp.bfloat16),
            pltpu.SemaphoreType.DMA((2,)),
            pltpu.SemaphoreType.DMA((2,)),
            pltpu.SemaphoreType.DMA((2,)),
            pltpu.SemaphoreType.DMA((2,)),
        ],
        compiler_params=pltpu.CompilerParams(
            collective_id=0,
            vmem_limit_bytes=128 * 1024 * 1024,
        ),
    )(x, W1, W2)


# device time: 96101 ns/iter; 2.4584x vs baseline; 1.0469x over previous
import jax
import jax.numpy as jnp
from jax import lax
from jax.experimental import pallas as pl
from jax.experimental.pallas import tpu as pltpu

N_DEV = 4


def kernel(x, W1, W2):
    W1 = W1.astype(jnp.bfloat16)
    W2 = W2.astype(jnp.bfloat16)
    m, _ = x.shape
    out_n = W2.shape[1]
    ch = m // N_DEV
    half = ch // 2

    def body(x_ref, w1_ref, w2_ref, out_ref,
             comm_r, comm_l,
             send_r, recv_r, send_l, recv_l):
        my_pos = lax.axis_index("i")
        left = (my_pos + N_DEV - 1) % N_DEV
        right = (my_pos + 1) % N_DEV

        def compute_half(c, off):
            xs = x_ref[pl.ds(c * ch + off, half), :].astype(jnp.bfloat16)
            h = jnp.dot(xs, w1_ref[...], preferred_element_type=jnp.float32)
            h = jnp.maximum(h, 0.0).astype(jnp.bfloat16)
            return jnp.dot(h, w2_ref[...], preferred_element_type=jnp.float32)

        comm_r[0, :, :] = compute_half(my_pos, 0).astype(jnp.bfloat16)
        comm_l[0, :, :] = compute_half(my_pos, half).astype(jnp.bfloat16)

        barrier_sem = pltpu.get_barrier_semaphore()
        for nbr in [left, right]:
            pl.semaphore_signal(
                barrier_sem, inc=1,
                device_id=(nbr,), device_id_type=pl.DeviceIdType.MESH,
            )
        pl.semaphore_wait(barrier_sem, 2)

        for k in range(N_DEV - 1):
            ss, rs = k % 2, (k + 1) % 2
            rdma_r = pltpu.make_async_remote_copy(
                src_ref=comm_r.at[ss], dst_ref=comm_r.at[rs],
                send_sem=send_r.at[ss], recv_sem=recv_r.at[rs],
                device_id=(right,), device_id_type=pl.DeviceIdType.MESH,
            )
            rdma_l = pltpu.make_async_remote_copy(
                src_ref=comm_l.at[ss], dst_ref=comm_l.at[rs],
                send_sem=send_l.at[ss], recv_sem=recv_l.at[rs],
                device_id=(left,), device_id_type=pl.DeviceIdType.MESH,
            )
            rdma_r.start()
            rdma_l.start()
            c_r = (my_pos + (N_DEV - 1 - k)) % N_DEV
            c_l = (my_pos + k + 1) % N_DEV
            p_r = compute_half(c_r, 0)
            p_l = compute_half(c_l, half)
            rdma_r.wait()
            rdma_l.wait()
            acc_r = comm_r[rs, :, :].astype(jnp.float32) + p_r
            acc_l = comm_l[rs, :, :].astype(jnp.float32) + p_l
            if k < N_DEV - 2:
                comm_r[rs, :, :] = acc_r.astype(jnp.bfloat16)
                comm_l[rs, :, :] = acc_l.astype(jnp.bfloat16)
            else:
                out_ref[pl.ds(c_r * ch, half), :] = acc_r.astype(jnp.bfloat16)
                out_ref[pl.ds(c_l * ch + half, half), :] = (
                    acc_l.astype(jnp.bfloat16))

        for t in range(N_DEV - 1):
            k = t + N_DEV - 1
            ss, rs = k % 2, (k + 1) % 2
            c_fr = (my_pos + 1 + N_DEV - t) % N_DEV
            c_fl = (my_pos + N_DEV - 1 + t) % N_DEV
            c_rr = (my_pos + N_DEV - t) % N_DEV
            c_rl = (my_pos + t) % N_DEV
            rdma_r = pltpu.make_async_remote_copy(
                src_ref=out_ref.at[pl.ds(c_fr * ch, half)],
                dst_ref=out_ref.at[pl.ds(c_fr * ch, half)],
                send_sem=send_r.at[ss], recv_sem=recv_r.at[rs],
                device_id=(right,), device_id_type=pl.DeviceIdType.MESH,
            )
            rdma_l = pltpu.make_async_remote_copy(
                src_ref=out_ref.at[pl.ds(c_fl * ch + half, half)],
                dst_ref=out_ref.at[pl.ds(c_fl * ch + half, half)],
                send_sem=send_l.at[ss], recv_sem=recv_l.at[rs],
                device_id=(left,), device_id_type=pl.DeviceIdType.MESH,
            )
            rdma_r.start()
            rdma_l.start()
            rdma_r.wait()
            rdma_l.wait()
            del c_rr, c_rl

    return pl.pallas_call(
        body,
        out_shape=jax.ShapeDtypeStruct((m, out_n), jnp.bfloat16),
        in_specs=[
            pl.BlockSpec(memory_space=pltpu.VMEM),
            pl.BlockSpec(memory_space=pltpu.VMEM),
            pl.BlockSpec(memory_space=pltpu.VMEM),
        ],
        out_specs=pl.BlockSpec(memory_space=pltpu.VMEM),
        scratch_shapes=[
            pltpu.VMEM((2, half, out_n), jnp.bfloat16),
            pltpu.VMEM((2, half, out_n), jnp.bfloat16),
            pltpu.SemaphoreType.DMA((2,)),
            pltpu.SemaphoreType.DMA((2,)),
            pltpu.SemaphoreType.DMA((2,)),
            pltpu.SemaphoreType.DMA((2,)),
        ],
        compiler_params=pltpu.CompilerParams(
            collective_id=0,
            vmem_limit_bytes=128 * 1024 * 1024,
        ),
    )(x, W1, W2)
